# baseline (speedup 1.0000x reference)
import functools

import jax
import jax.numpy as jnp
from jax import lax
from jax.experimental import pallas as pl
from jax.experimental.pallas import tpu as pltpu
from jax.experimental.pallas import tpu_sc as plsc

N_NODES = 10000
N_EDGES = 320000
DIM = 128
NPAD = 10240

NC = 2
NS = 16
NW = NC * NS
CHUNK = 112
NCHUNKS = 90
EDGES_PER_W = NCHUNKS * CHUNK
E_PAD = NW * EDGES_PER_W
ROWS_PER_S = NPAD // NS


def _sc_body(feat_hbm, src_hbm, dst_hbm, ew_hbm, zf_hbm, zd_hbm,
             acc_out, deg_out,
             acc, accd, src_a, dst_a, w_a, src_b, dst_b, w_b,
             rows_a, rows_b, ones_v, drain_v,
             isem_a, isem_b, gsem_a, gsem_b, ssem_a, ssem_b, dsem):
    c = lax.axis_index("c")
    s = lax.axis_index("s")
    wid = c * NS + s
    base = wid * NCHUNKS

    pltpu.sync_copy(zf_hbm.at[pl.ds(s * ROWS_PER_S, ROWS_PER_S)],
                    acc.at[pl.ds(s * ROWS_PER_S, ROWS_PER_S)])
    pltpu.sync_copy(zd_hbm.at[pl.ds(s * ROWS_PER_S, ROWS_PER_S)],
                    accd.at[pl.ds(s * ROWS_PER_S, ROWS_PER_S)])
    for i in range(CHUNK // 16):
        ones_v[pl.ds(i * 16, 16)] = jnp.full((16,), 1.0, jnp.float32)

    def idx_start(j, sv, dv, wv, sem):
        b = (base + jnp.minimum(j, NCHUNKS - 1)) * CHUNK
        pltpu.make_async_copy(src_hbm.at[pl.ds(b, CHUNK)], sv, sem).start()
        pltpu.make_async_copy(dst_hbm.at[pl.ds(b, CHUNK)], dv, sem).start()
        pltpu.make_async_copy(ew_hbm.at[pl.ds(b, CHUNK)], wv, sem).start()

    def idx_wait(sv, dv, wv, sem):
        pltpu.make_async_copy(src_hbm.at[pl.ds(0, CHUNK)], sv, sem).wait()
        pltpu.make_async_copy(dst_hbm.at[pl.ds(0, CHUNK)], dv, sem).wait()
        pltpu.make_async_copy(ew_hbm.at[pl.ds(0, CHUNK)], wv, sem).wait()

    def gstart(sv, buf, sem):
        pltpu.make_async_copy(feat_hbm.at[sv], buf, sem).start()

    def gwait(sv, buf, sem):
        pltpu.make_async_copy(feat_hbm.at[sv], buf, sem).wait()

    def swait(buf, ssem):
        pltpu.make_async_copy(buf, acc.at[dst_a], ssem).wait()

    def process(dv, wv, buf, ssem):
        pltpu.async_copy(ones_v, accd.at[dv], dsem, add=True)
        def scale_body(g, cc):
            w16 = wv[pl.ds(g * 16, 16)]
            for e in range(16):
                wval = w16[e]
                row = g * 16 + e
                for j in range(DIM // 16):
                    sl = pl.ds(j * 16, 16)
                    buf[row, sl] = buf[row, sl] * wval
            return cc

        lax.fori_loop(0, CHUNK // 16, scale_body, 0, unroll=True)
        pltpu.async_copy(buf, acc.at[dv], ssem, add=True)

    idx_start(0, src_a, dst_a, w_a, isem_a)
    idx_start(1, src_b, dst_b, w_b, isem_b)
    idx_wait(src_a, dst_a, w_a, isem_a)
    gstart(src_a, rows_a, gsem_a)

    def pipe_body(i, cc):
        k = i * 2
        gwait(src_a, rows_a, gsem_a)
        idx_wait(src_b, dst_b, w_b, isem_b)

        @pl.when(i > 0)
        def _():
            swait(rows_b, ssem_b)

        gstart(src_b, rows_b, gsem_b)
        process(dst_a, w_a, rows_a, ssem_a)
        idx_start(k + 2, src_a, dst_a, w_a, isem_a)
        gwait(src_b, rows_b, gsem_b)
        idx_wait(src_a, dst_a, w_a, isem_a)
        swait(rows_a, ssem_a)
        gstart(src_a, rows_a, gsem_a)
        process(dst_b, w_b, rows_b, ssem_b)
        idx_start(k + 3, src_b, dst_b, w_b, isem_b)
        return cc

    lax.fori_loop(0, NCHUNKS // 2, pipe_body, 0)
    gwait(src_a, rows_a, gsem_a)
    idx_wait(src_b, dst_b, w_b, isem_b)
    swait(rows_b, ssem_b)
    pltpu.make_async_copy(ew_hbm.at[pl.ds(0, NCHUNKS * CHUNK)], drain_v,
                          dsem).wait()
    plsc.subcore_barrier()

    sl = pl.ds(s * ROWS_PER_S, ROWS_PER_S)
    pltpu.sync_copy(acc.at[sl], acc_out.at[c, sl])
    pltpu.sync_copy(accd.at[sl], deg_out.at[c, sl])


_sc_aggregate = functools.partial(
    pl.kernel,
    out_type=(
        jax.ShapeDtypeStruct((NC, NPAD, DIM), jnp.float32),
        jax.ShapeDtypeStruct((NC, NPAD), jnp.float32),
    ),
    mesh=plsc.VectorSubcoreMesh(core_axis_name="c", subcore_axis_name="s"),
    scratch_types=[
        pltpu.VMEM_SHARED((NPAD, DIM), jnp.float32),
        pltpu.VMEM_SHARED((NPAD,), jnp.float32),
        pltpu.VMEM((CHUNK,), jnp.int32),
        pltpu.VMEM((CHUNK,), jnp.int32),
        pltpu.VMEM((CHUNK,), jnp.float32),
        pltpu.VMEM((CHUNK,), jnp.int32),
        pltpu.VMEM((CHUNK,), jnp.int32),
        pltpu.VMEM((CHUNK,), jnp.float32),
        pltpu.VMEM((CHUNK, DIM), jnp.float32),
        pltpu.VMEM((CHUNK, DIM), jnp.float32),
        pltpu.VMEM((CHUNK,), jnp.float32),
        pltpu.VMEM((NCHUNKS * CHUNK,), jnp.float32),
        pltpu.SemaphoreType.DMA,
        pltpu.SemaphoreType.DMA,
        pltpu.SemaphoreType.DMA,
        pltpu.SemaphoreType.DMA,
        pltpu.SemaphoreType.DMA,
        pltpu.SemaphoreType.DMA,
        pltpu.SemaphoreType.DMA,
    ],
)(_sc_body)


def _tc_body(feat_ref, acc_ref, deg_ref, ws_ref, wn_ref, b_ref, out_ref):
    f = feat_ref[...]
    sm = acc_ref[0] + acc_ref[1]
    deg = deg_ref[0] + deg_ref[1]
    h_neigh = sm / jnp.maximum(deg, 1.0)[:, None]
    dn = (((1,), (1,)), ((), ()))
    hn = lax.dot_general(h_neigh, wn_ref[...], dn,
                         preferred_element_type=jnp.float32)
    hs = lax.dot_general(f, ws_ref[...], dn,
                         preferred_element_type=jnp.float32)
    out_ref[...] = hs + hn + b_ref[...]


_TC_BLOCK = 512


def _tc_dense(features, accs, degs, w_self, w_neigh, bias2d):
    grid = (pl.cdiv(N_NODES, _TC_BLOCK),)
    return pl.pallas_call(
        _tc_body,
        grid=grid,
        in_specs=[
            pl.BlockSpec((_TC_BLOCK, DIM), lambda i: (i, 0)),
            pl.BlockSpec((NC, _TC_BLOCK, DIM), lambda i: (0, i, 0)),
            pl.BlockSpec((NC, _TC_BLOCK), lambda i: (0, i)),
            pl.BlockSpec((DIM, DIM), lambda i: (0, 0)),
            pl.BlockSpec((DIM, DIM), lambda i: (0, 0)),
            pl.BlockSpec((1, DIM), lambda i: (0, 0)),
        ],
        out_specs=pl.BlockSpec((_TC_BLOCK, DIM), lambda i: (i, 0)),
        out_shape=jax.ShapeDtypeStruct((N_NODES, DIM), jnp.float32),
    )(features, accs, degs, w_self, w_neigh, bias2d)


def kernel(features, edge_index, edge_weight, W_self, W_neigh, bias):
    npad_e = E_PAD - N_EDGES
    src = jnp.concatenate([edge_index[0].astype(jnp.int32),
                           jnp.zeros((npad_e,), jnp.int32)])
    dst = jnp.concatenate([edge_index[1].astype(jnp.int32),
                           jnp.full((npad_e,), N_NODES, jnp.int32)])
    ew = jnp.concatenate([edge_weight.astype(jnp.float32),
                          jnp.zeros((npad_e,), jnp.float32)])
    zf = jnp.zeros((NPAD, DIM), jnp.float32)
    zd = jnp.zeros((NPAD,), jnp.float32)
    accs, degs = _sc_aggregate(features, src, dst, ew, zf, zd)
    return _tc_dense(features, accs, degs, W_self, W_neigh,
                     bias.reshape(1, DIM))


# speedup vs baseline: 1.0047x; 1.0047x over previous
import functools

import jax
import jax.numpy as jnp
from jax import lax
from jax.experimental import pallas as pl
from jax.experimental.pallas import tpu as pltpu
from jax.experimental.pallas import tpu_sc as plsc

N_NODES = 10000
N_EDGES = 320000
DIM = 128
NPAD = 10240

NC = 2
NS = 16
NW = NC * NS
CHUNK = 96
NCHUNKS = 105
EDGES_PER_W = NCHUNKS * CHUNK
E_PAD = NW * EDGES_PER_W
ROWS_PER_S = NPAD // NS


def _sc_body(feat_hbm, src_hbm, dst_hbm, ew_hbm, zf_hbm, zd_hbm,
             acc_out, deg_out,
             acc, accd, src_a, dst_a, w_a, src_b, dst_b, w_b,
             rows_a, rows_b, ones_v, drain_v,
             isem_a, isem_b, gsem_a, gsem_b, ssem_a, ssem_b, dsem):
    c = lax.axis_index("c")
    s = lax.axis_index("s")
    wid = c * NS + s
    base = wid * NCHUNKS

    pltpu.sync_copy(zf_hbm.at[pl.ds(s * ROWS_PER_S, ROWS_PER_S)],
                    acc.at[pl.ds(s * ROWS_PER_S, ROWS_PER_S)])
    pltpu.sync_copy(zd_hbm.at[pl.ds(s * ROWS_PER_S, ROWS_PER_S)],
                    accd.at[pl.ds(s * ROWS_PER_S, ROWS_PER_S)])
    for i in range(CHUNK // 16):
        ones_v[pl.ds(i * 16, 16)] = jnp.full((16,), 1.0, jnp.float32)

    def idx_start(j, sv, dv, wv, sem):
        b = (base + jnp.minimum(j, NCHUNKS - 1)) * CHUNK
        pltpu.make_async_copy(src_hbm.at[pl.ds(b, CHUNK)], sv, sem).start()
        pltpu.make_async_copy(dst_hbm.at[pl.ds(b, CHUNK)], dv, sem).start()
        pltpu.make_async_copy(ew_hbm.at[pl.ds(b, CHUNK)], wv, sem).start()

    def idx_wait(sv, dv, wv, sem):
        pltpu.make_async_copy(src_hbm.at[pl.ds(0, CHUNK)], sv, sem).wait()
        pltpu.make_async_copy(dst_hbm.at[pl.ds(0, CHUNK)], dv, sem).wait()
        pltpu.make_async_copy(ew_hbm.at[pl.ds(0, CHUNK)], wv, sem).wait()

    def gstart(sv, buf, sem):
        pltpu.make_async_copy(feat_hbm.at[sv], buf, sem).start()

    def gwait(sv, buf, sem):
        pltpu.make_async_copy(feat_hbm.at[sv], buf, sem).wait()

    def swait(buf, ssem):
        pltpu.make_async_copy(buf, acc.at[dst_a], ssem).wait()

    def process(dv, wv, buf, ssem):
        pltpu.async_copy(ones_v, accd.at[dv], dsem, add=True)
        def scale_body(g, cc):
            w16 = wv[pl.ds(g * 16, 16)]
            for e in range(16):
                wval = w16[e]
                row = g * 16 + e
                for j in range(DIM // 16):
                    sl = pl.ds(j * 16, 16)
                    buf[row, sl] = buf[row, sl] * wval
            return cc

        lax.fori_loop(0, CHUNK // 16, scale_body, 0, unroll=True)
        pltpu.async_copy(buf, acc.at[dv], ssem, add=True)

    idx_start(0, src_a, dst_a, w_a, isem_a)
    idx_start(1, src_b, dst_b, w_b, isem_b)
    idx_wait(src_a, dst_a, w_a, isem_a)
    gstart(src_a, rows_a, gsem_a)

    def pipe_body(i, cc):
        k = i * 2
        gwait(src_a, rows_a, gsem_a)
        idx_wait(src_b, dst_b, w_b, isem_b)

        @pl.when(i > 0)
        def _():
            swait(rows_b, ssem_b)

        gstart(src_b, rows_b, gsem_b)
        process(dst_a, w_a, rows_a, ssem_a)
        idx_start(k + 2, src_a, dst_a, w_a, isem_a)
        gwait(src_b, rows_b, gsem_b)
        idx_wait(src_a, dst_a, w_a, isem_a)
        swait(rows_a, ssem_a)
        gstart(src_a, rows_a, gsem_a)
        process(dst_b, w_b, rows_b, ssem_b)
        idx_start(k + 3, src_b, dst_b, w_b, isem_b)
        return cc

    lax.fori_loop(0, (NCHUNKS - 1) // 2, pipe_body, 0)
    gwait(src_a, rows_a, gsem_a)
    idx_wait(src_b, dst_b, w_b, isem_b)
    swait(rows_b, ssem_b)
    process(dst_a, w_a, rows_a, ssem_a)
    swait(rows_a, ssem_a)
    pltpu.make_async_copy(ew_hbm.at[pl.ds(0, NCHUNKS * CHUNK)], drain_v,
                          dsem).wait()
    plsc.subcore_barrier()

    sl = pl.ds(s * ROWS_PER_S, ROWS_PER_S)
    pltpu.sync_copy(acc.at[sl], acc_out.at[c, sl])
    pltpu.sync_copy(accd.at[sl], deg_out.at[c, sl])


_sc_aggregate = functools.partial(
    pl.kernel,
    out_type=(
        jax.ShapeDtypeStruct((NC, NPAD, DIM), jnp.float32),
        jax.ShapeDtypeStruct((NC, NPAD), jnp.float32),
    ),
    mesh=plsc.VectorSubcoreMesh(core_axis_name="c", subcore_axis_name="s"),
    scratch_types=[
        pltpu.VMEM_SHARED((NPAD, DIM), jnp.float32),
        pltpu.VMEM_SHARED((NPAD,), jnp.float32),
        pltpu.VMEM((CHUNK,), jnp.int32),
        pltpu.VMEM((CHUNK,), jnp.int32),
        pltpu.VMEM((CHUNK,), jnp.float32),
        pltpu.VMEM((CHUNK,), jnp.int32),
        pltpu.VMEM((CHUNK,), jnp.int32),
        pltpu.VMEM((CHUNK,), jnp.float32),
        pltpu.VMEM((CHUNK, DIM), jnp.float32),
        pltpu.VMEM((CHUNK, DIM), jnp.float32),
        pltpu.VMEM((CHUNK,), jnp.float32),
        pltpu.VMEM((NCHUNKS * CHUNK,), jnp.float32),
        pltpu.SemaphoreType.DMA,
        pltpu.SemaphoreType.DMA,
        pltpu.SemaphoreType.DMA,
        pltpu.SemaphoreType.DMA,
        pltpu.SemaphoreType.DMA,
        pltpu.SemaphoreType.DMA,
        pltpu.SemaphoreType.DMA,
    ],
)(_sc_body)


def _tc_body(feat_ref, acc_ref, deg_ref, ws_ref, wn_ref, b_ref, out_ref):
    f = feat_ref[...]
    sm = acc_ref[0] + acc_ref[1]
    deg = deg_ref[0] + deg_ref[1]
    h_neigh = sm / jnp.maximum(deg, 1.0)[:, None]
    dn = (((1,), (1,)), ((), ()))
    hn = lax.dot_general(h_neigh, wn_ref[...], dn,
                         preferred_element_type=jnp.float32)
    hs = lax.dot_general(f, ws_ref[...], dn,
                         preferred_element_type=jnp.float32)
    out_ref[...] = hs + hn + b_ref[...]


_TC_BLOCK = 512


def _tc_dense(features, accs, degs, w_self, w_neigh, bias2d):
    grid = (pl.cdiv(N_NODES, _TC_BLOCK),)
    return pl.pallas_call(
        _tc_body,
        grid=grid,
        in_specs=[
            pl.BlockSpec((_TC_BLOCK, DIM), lambda i: (i, 0)),
            pl.BlockSpec((NC, _TC_BLOCK, DIM), lambda i: (0, i, 0)),
            pl.BlockSpec((NC, _TC_BLOCK), lambda i: (0, i)),
            pl.BlockSpec((DIM, DIM), lambda i: (0, 0)),
            pl.BlockSpec((DIM, DIM), lambda i: (0, 0)),
            pl.BlockSpec((1, DIM), lambda i: (0, 0)),
        ],
        out_specs=pl.BlockSpec((_TC_BLOCK, DIM), lambda i: (i, 0)),
        out_shape=jax.ShapeDtypeStruct((N_NODES, DIM), jnp.float32),
    )(features, accs, degs, w_self, w_neigh, bias2d)


def kernel(features, edge_index, edge_weight, W_self, W_neigh, bias):
    npad_e = E_PAD - N_EDGES
    src = jnp.concatenate([edge_index[0].astype(jnp.int32),
                           jnp.zeros((npad_e,), jnp.int32)])
    dst = jnp.concatenate([edge_index[1].astype(jnp.int32),
                           jnp.full((npad_e,), N_NODES, jnp.int32)])
    ew = jnp.concatenate([edge_weight.astype(jnp.float32),
                          jnp.zeros((npad_e,), jnp.float32)])
    zf = jnp.zeros((NPAD, DIM), jnp.float32)
    zd = jnp.zeros((NPAD,), jnp.float32)
    accs, degs = _sc_aggregate(features, src, dst, ew, zf, zd)
    return _tc_dense(features, accs, degs, W_self, W_neigh,
                     bias.reshape(1, DIM))


# speedup vs baseline: 1.5445x; 1.5372x over previous
import functools

import jax
import jax.numpy as jnp
from jax import lax
from jax.experimental import pallas as pl
from jax.experimental.pallas import tpu as pltpu
from jax.experimental.pallas import tpu_sc as plsc

N_NODES = 10000
N_EDGES = 320000
DIM = 128
NPAD = 10240

NC = 2
NS = 16
NW = NC * NS
EDGES_PER_W = N_EDGES // NW
CHUNK = 80
NCHUNKS = EDGES_PER_W // CHUNK
ROWS_PER_S = NPAD // NS


def _sc_body(feat_hbm, src_hbm, dst_hbm, ew_hbm, zf_hbm, zd_hbm,
             acc_out, deg_out,
             acc, accd, src_a, dst_a, w_a, src_b, dst_b, w_b,
             rows_a, rows_b, ones_v, drain_v,
             isem_a, isem_b, gsem_a, gsem_b, ssem_a, ssem_b, dsem):
    c = lax.axis_index("c")
    s = lax.axis_index("s")
    wid = c * NS + s
    base = wid * NCHUNKS

    pltpu.sync_copy(zf_hbm.at[pl.ds(s * ROWS_PER_S, ROWS_PER_S)],
                    acc.at[pl.ds(s * ROWS_PER_S, ROWS_PER_S)])
    pltpu.sync_copy(zd_hbm.at[pl.ds(s * ROWS_PER_S, ROWS_PER_S)],
                    accd.at[pl.ds(s * ROWS_PER_S, ROWS_PER_S)])
    for i in range(CHUNK // 16):
        ones_v[pl.ds(i * 16, 16)] = jnp.full((16,), 1.0, jnp.float32)

    def idx_start(j, sv, dv, wv, sem):
        b = (base + jnp.minimum(j, NCHUNKS - 1)) * CHUNK
        pltpu.make_async_copy(src_hbm.at[pl.ds(b, CHUNK)], sv, sem).start()
        pltpu.make_async_copy(dst_hbm.at[pl.ds(b, CHUNK)], dv, sem).start()
        pltpu.make_async_copy(ew_hbm.at[pl.ds(b, CHUNK)], wv, sem).start()

    def idx_wait(sv, dv, wv, sem):
        pltpu.make_async_copy(src_hbm.at[pl.ds(0, CHUNK)], sv, sem).wait()
        pltpu.make_async_copy(dst_hbm.at[pl.ds(0, CHUNK)], dv, sem).wait()
        pltpu.make_async_copy(ew_hbm.at[pl.ds(0, CHUNK)], wv, sem).wait()

    def gstart(sv, buf, sem):
        pltpu.make_async_copy(feat_hbm.at[sv], buf, sem).start()

    def gwait(sv, buf, sem):
        pltpu.make_async_copy(feat_hbm.at[sv], buf, sem).wait()

    def swait(buf, ssem):
        pltpu.make_async_copy(buf, acc.at[dst_a], ssem).wait()

    def process(dv, wv, buf, ssem):
        pltpu.async_copy(ones_v, accd.at[dv], dsem, add=True)
        def scale_body(g, cc):
            w16 = wv[pl.ds(g * 16, 16)]
            for e in range(16):
                wval = w16[e]
                row = g * 16 + e
                for j in range(DIM // 16):
                    sl = pl.ds(j * 16, 16)
                    buf[row, sl] = buf[row, sl] * wval
            return cc

        lax.fori_loop(0, CHUNK // 16, scale_body, 0)
        pltpu.async_copy(buf, acc.at[dv], ssem, add=True)

    idx_start(0, src_a, dst_a, w_a, isem_a)
    idx_start(1, src_b, dst_b, w_b, isem_b)
    idx_wait(src_a, dst_a, w_a, isem_a)
    gstart(src_a, rows_a, gsem_a)

    def pipe_body(i, cc):
        k = i * 2
        gwait(src_a, rows_a, gsem_a)
        idx_wait(src_b, dst_b, w_b, isem_b)

        @pl.when(i > 0)
        def _():
            swait(rows_b, ssem_b)

        gstart(src_b, rows_b, gsem_b)
        process(dst_a, w_a, rows_a, ssem_a)
        idx_start(k + 2, src_a, dst_a, w_a, isem_a)
        gwait(src_b, rows_b, gsem_b)
        idx_wait(src_a, dst_a, w_a, isem_a)
        swait(rows_a, ssem_a)
        gstart(src_a, rows_a, gsem_a)
        process(dst_b, w_b, rows_b, ssem_b)
        idx_start(k + 3, src_b, dst_b, w_b, isem_b)
        return cc

    lax.fori_loop(0, (NCHUNKS - 1) // 2, pipe_body, 0)
    gwait(src_a, rows_a, gsem_a)
    idx_wait(src_b, dst_b, w_b, isem_b)
    swait(rows_b, ssem_b)
    process(dst_a, w_a, rows_a, ssem_a)
    swait(rows_a, ssem_a)
    pltpu.make_async_copy(ew_hbm.at[pl.ds(0, NCHUNKS * CHUNK)], drain_v,
                          dsem).wait()
    plsc.subcore_barrier()

    sl = pl.ds(s * ROWS_PER_S, ROWS_PER_S)
    pltpu.sync_copy(acc.at[sl], acc_out.at[c, sl])
    pltpu.sync_copy(accd.at[sl], deg_out.at[c, sl])


_sc_aggregate = functools.partial(
    pl.kernel,
    out_type=(
        jax.ShapeDtypeStruct((NC, NPAD, DIM), jnp.float32),
        jax.ShapeDtypeStruct((NC, NPAD), jnp.float32),
    ),
    mesh=plsc.VectorSubcoreMesh(core_axis_name="c", subcore_axis_name="s"),
    scratch_types=[
        pltpu.VMEM_SHARED((NPAD, DIM), jnp.float32),
        pltpu.VMEM_SHARED((NPAD,), jnp.float32),
        pltpu.VMEM((CHUNK,), jnp.int32),
        pltpu.VMEM((CHUNK,), jnp.int32),
        pltpu.VMEM((CHUNK,), jnp.float32),
        pltpu.VMEM((CHUNK,), jnp.int32),
        pltpu.VMEM((CHUNK,), jnp.int32),
        pltpu.VMEM((CHUNK,), jnp.float32),
        pltpu.VMEM((CHUNK, DIM), jnp.float32),
        pltpu.VMEM((CHUNK, DIM), jnp.float32),
        pltpu.VMEM((CHUNK,), jnp.float32),
        pltpu.VMEM((NCHUNKS * CHUNK,), jnp.float32),
        pltpu.SemaphoreType.DMA,
        pltpu.SemaphoreType.DMA,
        pltpu.SemaphoreType.DMA,
        pltpu.SemaphoreType.DMA,
        pltpu.SemaphoreType.DMA,
        pltpu.SemaphoreType.DMA,
        pltpu.SemaphoreType.DMA,
    ],
)(_sc_body)


def _tc_body(feat_ref, acc_ref, deg_ref, ws_ref, wn_ref, b_ref, out_ref):
    f = feat_ref[...]
    sm = acc_ref[0] + acc_ref[1]
    deg = deg_ref[0] + deg_ref[1]
    h_neigh = sm / jnp.maximum(deg, 1.0)[:, None]
    dn = (((1,), (1,)), ((), ()))
    hn = lax.dot_general(h_neigh, wn_ref[...], dn,
                         preferred_element_type=jnp.float32)
    hs = lax.dot_general(f, ws_ref[...], dn,
                         preferred_element_type=jnp.float32)
    out_ref[...] = hs + hn + b_ref[...]


_TC_BLOCK = 512


def _tc_dense(features, accs, degs, w_self, w_neigh, bias2d):
    grid = (pl.cdiv(N_NODES, _TC_BLOCK),)
    return pl.pallas_call(
        _tc_body,
        grid=grid,
        in_specs=[
            pl.BlockSpec((_TC_BLOCK, DIM), lambda i: (i, 0)),
            pl.BlockSpec((NC, _TC_BLOCK, DIM), lambda i: (0, i, 0)),
            pl.BlockSpec((NC, _TC_BLOCK), lambda i: (0, i)),
            pl.BlockSpec((DIM, DIM), lambda i: (0, 0)),
            pl.BlockSpec((DIM, DIM), lambda i: (0, 0)),
            pl.BlockSpec((1, DIM), lambda i: (0, 0)),
        ],
        out_specs=pl.BlockSpec((_TC_BLOCK, DIM), lambda i: (i, 0)),
        out_shape=jax.ShapeDtypeStruct((N_NODES, DIM), jnp.float32),
    )(features, accs, degs, w_self, w_neigh, bias2d)


def kernel(features, edge_index, edge_weight, W_self, W_neigh, bias):
    src = edge_index[0].astype(jnp.int32)
    dst = edge_index[1].astype(jnp.int32)
    ew = edge_weight.astype(jnp.float32)
    zf = jnp.zeros((NPAD, DIM), jnp.float32)
    zd = jnp.zeros((NPAD,), jnp.float32)
    accs, degs = _sc_aggregate(features, src, dst, ew, zf, zd)
    return _tc_dense(features, accs, degs, W_self, W_neigh,
                     bias.reshape(1, DIM))
